# Initial kernel scaffold; baseline (speedup 1.0000x reference)
import jax
import jax.numpy as jnp
from jax.experimental import pallas as pl


def kernel(x, grids, G, H, size):
    raise NotImplementedError("write your pallas kernel here")


# speedup vs baseline: 496.7728x; 496.7728x over previous
import functools

import jax
import jax.numpy as jnp
from jax import lax
from jax.experimental import pallas as pl
from jax.experimental.pallas import tpu as pltpu
from jax.experimental.pallas import tpu_sc as plsc

IN_DIM = 3
M = 3
N = 32
C = 2
W = 4096
PAD_L = 2
TW = W + 8

BB = 8192
NG = 4
NL = N // NG
NR = 8
CR = 1024


_INV2PI = 0.15915494309189535
_MAGIC = 12582912.0
_C1 = 6.28125
_C2 = 0.0019353071795864769
_SIN_P = (
    9.9999999420e-01,
    -1.6666664500e-01,
    8.3333096487e-03,
    -1.9840126801e-04,
    2.7528926525e-06,
    -2.4672325863e-08,
    1.3435869084e-10,
)


def _fast_sin(a):
    n = a * _INV2PI + _MAGIC - _MAGIC
    r = a - n * _C1 - n * _C2
    r2 = r * r
    p = _SIN_P[6]
    for k in (5, 4, 3, 2, 1, 0):
        p = p * r2 + _SIN_P[k]
    return p * r


def _coord_body(xt_ref, g_ref, h_ref, o_ref):
    xt = xt_ref[...]
    g = g_ref[...]
    h = h_ref[...]
    a = h + g[:, 0:1] * xt[0:1, :]
    a += g[:, 1:2] * xt[1:2, :]
    a += g[:, 2:3] * xt[2:3, :]
    s = _fast_sin(a)
    gx = s[0:N, :] * s[N : 2 * N, :] * s[2 * N : 3 * N, :]
    o_ref[...] = ((gx + 1.0) * W - 1.0) * 0.5


def _coords(xt, gmt, hcol, bn):
    return pl.pallas_call(
        _coord_body,
        grid=(bn // BB,),
        in_specs=[
            pl.BlockSpec((IN_DIM, BB), lambda i: (0, i)),
            pl.BlockSpec((M * N, IN_DIM), lambda i: (0, 0)),
            pl.BlockSpec((M * N, 1), lambda i: (0, 0)),
        ],
        out_specs=pl.BlockSpec((N, BB), lambda i: (0, i)),
        out_shape=jax.ShapeDtypeStruct((N, bn), jnp.float32),
    )(xt, gmt, hcol)


def _interp_call(ix_t, tab, bn):
    rt = bn // NR
    mesh = plsc.VectorSubcoreMesh(
        core_axis_name="c", subcore_axis_name="s", num_cores=2, num_subcores=16
    )

    @functools.partial(
        pl.kernel,
        out_type=jax.ShapeDtypeStruct((bn, N * C), jnp.float32),
        mesh=mesh,
        scratch_types=[
            pltpu.VMEM((NL, TW * C), jnp.float32),
            pltpu.VMEM((NL, CR), jnp.float32),
            pltpu.VMEM((CR, NL * C), jnp.float32),
        ],
        compiler_params=pltpu.CompilerParams(
            use_tc_tiling_on_sc=False, needs_layout_passes=False
        ),
    )
    def run(ix_hbm, tab_hbm, out_hbm, tab_v, ix_v, out_v):
        wid = lax.axis_index("s") * 2 + lax.axis_index("c")
        grp = wid % NG
        rng = wid // NG
        rows0 = rng * rt
        pltpu.sync_copy(tab_hbm.at[pl.ds(grp * NL, NL)], tab_v)

        lane = lax.iota(jnp.int32, 16)
        lvl = lane & (NL - 1)
        rowoff = lane >> 3
        ch0 = lvl * C

        @pl.loop(0, rt, step=CR)
        def _chunk(cb):
            pltpu.sync_copy(
                ix_hbm.at[pl.ds(grp * NL, NL), pl.ds(rows0 + cb, CR)], ix_v
            )

            @pl.loop(0, CR, step=2)
            def _vec(v):
                rowv = rowoff + v
                ix = plsc.load_gather(ix_v, [lvl, rowv])
                ixp1 = ix + 1.0
                base = ixp1.astype(jnp.int32)
                t = ixp1 - base.astype(jnp.float32)
                t2 = t * t
                t3 = t2 * t
                w0 = -0.75 * (t3 - 2.0 * t2 + t)
                w3 = -0.75 * (t2 - t3)
                w1 = 1.25 * t3 - 2.25 * t2 + 1.0
                w2 = 1.0 - w0 - w1 - w3
                base2 = base * 2
                v00 = plsc.load_gather(tab_v, [lvl, base2])
                v01 = plsc.load_gather(tab_v, [lvl, base2 + 1])
                v10 = plsc.load_gather(tab_v, [lvl, base2 + 2])
                v11 = plsc.load_gather(tab_v, [lvl, base2 + 3])
                v20 = plsc.load_gather(tab_v, [lvl, base2 + 4])
                v21 = plsc.load_gather(tab_v, [lvl, base2 + 5])
                v30 = plsc.load_gather(tab_v, [lvl, base2 + 6])
                v31 = plsc.load_gather(tab_v, [lvl, base2 + 7])
                acc0 = w0 * v00 + w1 * v10 + w2 * v20 + w3 * v30
                acc1 = w0 * v01 + w1 * v11 + w2 * v21 + w3 * v31
                plsc.store_scatter(out_v, [rowv, ch0], acc0)
                plsc.store_scatter(out_v, [rowv, ch0 + 1], acc1)

            pltpu.sync_copy(
                out_v,
                out_hbm.at[pl.ds(rows0 + cb, CR), pl.ds(grp * NL * C, NL * C)],
            )

    return run(ix_t, tab)


def kernel(x, grids, G, H, size):
    bn = x.shape[0]
    xt = x.T
    gmt = G.reshape(IN_DIM, M * N).T
    hcol = H.reshape(M * N, 1)
    tab = jnp.pad(
        jnp.transpose(grids, (0, 2, 1)), ((0, 0), (PAD_L, TW - W - PAD_L), (0, 0))
    ).reshape(N, TW * C)
    ix_t = _coords(xt, gmt, hcol, bn)
    return _interp_call(ix_t, tab, bn)
